# baseline (speedup 1.0000x reference)
import functools

import jax
import jax.numpy as jnp
from jax import lax
from jax.experimental import pallas as pl
from jax.experimental.pallas import tpu as pltpu
from jax.experimental.pallas import tpu_sc as plsc

N_NODES = 10000
D_FEAT = 128
K_STEPS = 2
NC = 2
NS = 16
NW = NC * NS
N_PAD = 10240
HALF = N_PAD // NC
PAD_ROWS = N_PAD - N_NODES
ROWS_PER_TILE = N_PAD // NW
ROWS_PER_SUB = N_PAD // NS
SUB_ROWS = HALF // NS
CHUNK = 128
LANES = 16
FILL = 384

_MESH = plsc.VectorSubcoreMesh(core_axis_name="c", subcore_axis_name="s")
_f32 = jnp.float32
_i32 = jnp.int32
_PARAMS = pltpu.CompilerParams(use_tc_tiling_on_sc=False)
_PARAMS_NOLAYOUT = pltpu.CompilerParams(
    use_tc_tiling_on_sc=False, needs_layout_passes=False)


def _part_body(ept, ei_h, ones_h, zeros16_h,
               rows_l, cols_l, counts, degp,
               row1d, col1d, ra, ca, rb, cb, ones_v, cnt_v, deg_acc, sem_d):
    cid = lax.axis_index("c")
    sid = lax.axis_index("s")
    wid = cid * NS + sid
    esl = pl.ds(wid * ept, ept)
    pltpu.sync_copy(ei_h.at[0, esl], row1d)
    pltpu.sync_copy(ei_h.at[1, esl], col1d)
    pltpu.sync_copy(ones_h, ones_v)
    sl = pl.ds(sid * ROWS_PER_SUB, ROWS_PER_SUB)
    pltpu.sync_copy(zeros16_h, deg_acc.at[sl])
    plsc.subcore_barrier()

    nfull = ept // CHUNK
    ntail = (ept % CHUNK) // LANES

    def _fire_d(j):
        pltpu.async_copy(ones_v, deg_acc.at[col1d.at[pl.ds(j * CHUNK, CHUNK)]],
                         sem_d, add=True)

    def _wait_d():
        pltpu.make_async_copy(
            ones_v, deg_acc.at[col1d.at[pl.ds(0, CHUNK)]], sem_d).wait()

    _fire_d(0)
    _fire_d(1)

    def dstep(j, carry):
        _fire_d(j + 2)
        _wait_d()
        return carry

    lax.fori_loop(0, nfull - 2, dstep, 0)
    for t in range(ntail):
        pltpu.async_copy(
            ones_v.at[pl.ds(0, LANES)],
            deg_acc.at[col1d.at[pl.ds(nfull * CHUNK + t * LANES, LANES)]],
            sem_d, add=True)

    def cstep(k, carry):
        offa, offb = carry
        r16 = row1d[pl.ds(k * LANES, LANES)]
        c16 = col1d[pl.ds(k * LANES, LANES)]
        ma = r16 < HALF
        mb = jnp.logical_not(ma)
        pca = plsc.cumsum(ma.astype(_i32))
        posa = jnp.full((LANES,), offa - 1, _i32) + pca
        posb = jnp.full((LANES,), offb, _i32) + (lax.iota(_i32, LANES) - pca)
        plsc.store_scatter(ra, [posa], r16, mask=ma)
        plsc.store_scatter(ca, [posa], c16, mask=ma)
        plsc.store_scatter(rb, [posb], r16 - HALF, mask=mb)
        plsc.store_scatter(cb, [posb], c16, mask=mb)
        na = pca[LANES - 1]
        return offa + na, offb + (LANES - na)

    offa, offb = lax.fori_loop(0, ept // LANES, cstep, (0, 0))
    _wait_d()
    _wait_d()
    for t in range(ntail):
        pltpu.make_async_copy(
            ones_v.at[pl.ds(0, LANES)],
            deg_acc.at[col1d.at[pl.ds(0, LANES)]], sem_d).wait()

    iota = lax.iota(_i32, LANES)
    for k in range(FILL // LANES):
        rd = (iota * 331 + k * LANES) % HALF
        cd = N_NODES + ((iota + k) * 7) % PAD_ROWS
        ra[pl.ds(offa + k * LANES, LANES)] = rd
        ca[pl.ds(offa + k * LANES, LANES)] = cd
        rb[pl.ds(offb + k * LANES, LANES)] = rd
        cb[pl.ds(offb + k * LANES, LANES)] = cd

    def _chunks(off):
        c = (off + CHUNK - 1) // CHUNK
        c = c + (c % 2)
        return jnp.maximum(c, 2)

    cnt_v[...] = jnp.full((LANES,), _chunks(offa), _i32)
    pltpu.sync_copy(cnt_v, counts.at[0, wid])
    cnt_v[...] = jnp.full((LANES,), _chunks(offb), _i32)
    pltpu.sync_copy(cnt_v, counts.at[1, wid])
    pltpu.sync_copy(ra, rows_l.at[0, wid])
    pltpu.sync_copy(ca, cols_l.at[0, wid])
    pltpu.sync_copy(rb, rows_l.at[1, wid])
    pltpu.sync_copy(cb, cols_l.at[1, wid])

    plsc.subcore_barrier()
    pltpu.sync_copy(deg_acc.at[sl], degp.at[cid, sl])


def _make_part(ept):
    cap = ept + FILL
    return pl.kernel(
        functools.partial(_part_body, ept),
        out_type=(
            jax.ShapeDtypeStruct((NC, NW, cap), _i32),
            jax.ShapeDtypeStruct((NC, NW, cap), _i32),
            jax.ShapeDtypeStruct((NC, NW, LANES), _i32),
            jax.ShapeDtypeStruct((NC, N_PAD, LANES), _f32),
        ),
        mesh=_MESH,
        compiler_params=_PARAMS_NOLAYOUT,
        scratch_types=[
            pltpu.VMEM((ept,), _i32),
            pltpu.VMEM((ept,), _i32),
            pltpu.VMEM((cap,), _i32),
            pltpu.VMEM((cap,), _i32),
            pltpu.VMEM((cap,), _i32),
            pltpu.VMEM((cap,), _i32),
            pltpu.VMEM((CHUNK, LANES), _f32),
            pltpu.VMEM((LANES,), _i32),
            pltpu.VMEM_SHARED((N_PAD, LANES), _f32),
            pltpu.SemaphoreType.DMA,
        ],
    )


def _scale_body(x_h, degp_h, zrows_h, zeros16_h, g_h, inv_h,
                x_v, da_v, db_v, g_v, inv_v):
    wid = lax.axis_index("c") * NS + lax.axis_index("s")
    r0 = jnp.minimum(wid * ROWS_PER_TILE, N_NODES - ROWS_PER_TILE)
    sl = pl.ds(r0, ROWS_PER_TILE)
    pltpu.sync_copy(x_h.at[sl], x_v)
    pltpu.sync_copy(degp_h.at[0, sl], da_v)
    pltpu.sync_copy(degp_h.at[1, sl], db_v)

    def row(r, carry):
        d16 = da_v[r, :] + db_v[r, :] + 1.0
        inv16 = 1.0 / d16
        inv_v[r, :] = inv16
        for f in range(D_FEAT // LANES):
            s = pl.ds(f * LANES, LANES)
            g_v[r, s] = x_v[r, s] * inv16
        return carry

    lax.fori_loop(0, ROWS_PER_TILE, row, 0)
    pltpu.sync_copy(g_v, g_h.at[sl])
    pltpu.sync_copy(inv_v, inv_h.at[sl])

    @pl.when(wid == NW - 1)
    def _():
        pltpu.sync_copy(zrows_h, g_h.at[pl.ds(N_NODES, PAD_ROWS)])
        pltpu.sync_copy(zeros16_h.at[pl.ds(0, PAD_ROWS)],
                        inv_h.at[pl.ds(N_NODES, PAD_ROWS)])


_scale_kernel = pl.kernel(
    _scale_body,
    out_type=(
        jax.ShapeDtypeStruct((N_PAD, D_FEAT), _f32),
        jax.ShapeDtypeStruct((N_PAD, LANES), _f32),
    ),
    mesh=_MESH,
    compiler_params=_PARAMS,
    name="scale_kernel",
    scratch_types=[
        pltpu.VMEM((ROWS_PER_TILE, D_FEAT), _f32),
        pltpu.VMEM((ROWS_PER_TILE, LANES), _f32),
        pltpu.VMEM((ROWS_PER_TILE, LANES), _f32),
        pltpu.VMEM((ROWS_PER_TILE, D_FEAT), _f32),
        pltpu.VMEM((ROWS_PER_TILE, LANES), _f32),
    ],
)


def _step_body(scale_out, g_h, rows_l, cols_l, counts_h, inv_h, out_h,
               ra, ca, rb, cb, cnt0_v, cnt1_v, inv_v, buf0, buf1, acc,
               s0, s1):
    cid = lax.axis_index("c")
    sid = lax.axis_index("s")
    half0 = cid * HALF
    lsl = pl.ds(sid * SUB_ROWS, SUB_ROWS)
    gsl = pl.ds(half0 + sid * SUB_ROWS, SUB_ROWS)
    pltpu.sync_copy(g_h.at[gsl], acc.at[lsl])
    src0 = sid
    src1 = NS + sid
    pltpu.sync_copy(rows_l.at[cid, src0], ra)
    pltpu.sync_copy(cols_l.at[cid, src0], ca)
    pltpu.sync_copy(rows_l.at[cid, src1], rb)
    pltpu.sync_copy(cols_l.at[cid, src1], cb)
    pltpu.sync_copy(counts_h.at[cid, src0], cnt0_v)
    pltpu.sync_copy(counts_h.at[cid, src1], cnt1_v)
    plsc.subcore_barrier()

    for r_l, c_l, cnt_ref in ((ra, ca, cnt0_v), (rb, cb, cnt1_v)):
        cnt = cnt_ref[...][0]

        def _idx(j):
            return c_l.at[pl.ds(j * CHUNK, CHUNK)]

        def _scatter(j, buf):
            pltpu.sync_copy(buf, acc.at[r_l.at[pl.ds(j * CHUNK, CHUNK)]],
                            add=True)

        pltpu.async_copy(g_h.at[_idx(0)], buf0, s0)
        pltpu.async_copy(g_h.at[_idx(1)], buf1, s1)

        def pair(i, carry):
            j0 = i * 2
            pltpu.make_async_copy(g_h.at[_idx(j0)], buf0, s0).wait()
            _scatter(j0, buf0)
            pltpu.async_copy(g_h.at[_idx(j0 + 2)], buf0, s0)
            pltpu.make_async_copy(g_h.at[_idx(j0 + 1)], buf1, s1).wait()
            _scatter(j0 + 1, buf1)
            pltpu.async_copy(g_h.at[_idx(j0 + 3)], buf1, s1)
            return carry

        lax.fori_loop(0, cnt // 2 - 1, pair, 0)
        pltpu.make_async_copy(g_h.at[_idx(cnt - 2)], buf0, s0).wait()
        _scatter(cnt - 2, buf0)
        pltpu.make_async_copy(g_h.at[_idx(cnt - 1)], buf1, s1).wait()
        _scatter(cnt - 1, buf1)

    plsc.subcore_barrier()
    if not scale_out:
        pltpu.sync_copy(acc.at[lsl], out_h.at[gsl])
    else:
        pltpu.sync_copy(inv_h.at[gsl], inv_v)
        done = 0
        while done < SUB_ROWS:
            cn = min(CHUNK, SUB_ROWS - done)
            pltpu.sync_copy(acc.at[pl.ds(sid * SUB_ROWS + done, cn)],
                            buf0.at[pl.ds(0, cn)])

            def rower(r, carry):
                inv16 = inv_v[done + r, :]
                for f in range(D_FEAT // LANES):
                    s = pl.ds(f * LANES, LANES)
                    buf0[r, s] = buf0[r, s] * inv16
                return carry

            lax.fori_loop(0, cn, rower, 0)
            pltpu.sync_copy(
                buf0.at[pl.ds(0, cn)],
                out_h.at[pl.ds(half0 + sid * SUB_ROWS + done, cn)])
            done += cn


def _make_step(ept, scale_out):
    cap = ept + FILL
    return pl.kernel(
        functools.partial(_step_body, scale_out),
        out_type=jax.ShapeDtypeStruct((N_PAD, D_FEAT), _f32),
        mesh=_MESH,
        compiler_params=_PARAMS,
        scratch_types=[
            pltpu.VMEM((cap,), _i32),
            pltpu.VMEM((cap,), _i32),
            pltpu.VMEM((cap,), _i32),
            pltpu.VMEM((cap,), _i32),
            pltpu.VMEM((LANES,), _i32),
            pltpu.VMEM((LANES,), _i32),
            pltpu.VMEM((SUB_ROWS, LANES), _f32),
            pltpu.VMEM((CHUNK, D_FEAT), _f32),
            pltpu.VMEM((CHUNK, D_FEAT), _f32),
            pltpu.VMEM_SHARED((HALF, D_FEAT), _f32),
            pltpu.SemaphoreType.DMA,
            pltpu.SemaphoreType.DMA,
        ],
    )


def kernel(x, edge_index):
    n, d = x.shape
    e = edge_index.shape[1]
    assert n == N_NODES and d == D_FEAT

    ei = edge_index.astype(_i32)

    align = NW * LANES
    if e % align:
        flat_pad = align - e % align
        dummy = N_NODES + (jnp.arange(flat_pad, dtype=_i32) % PAD_ROWS)
        ei = jnp.concatenate([ei, jnp.stack([dummy, dummy])], axis=1)
        e += flat_pad
    ept = e // NW

    zeros16 = jnp.zeros((ROWS_PER_SUB, LANES), _f32)
    zrows = jnp.zeros((PAD_ROWS, D_FEAT), _f32)
    ones = jnp.ones((CHUNK, LANES), _f32)

    rows_l, cols_l, counts, degp = _make_part(ept)(ei, ones, zeros16)
    g, inv = _scale_kernel(x.astype(_f32), degp, zrows, zeros16)
    h = None
    for k in range(K_STEPS):
        step = _make_step(ept, scale_out=(k < K_STEPS - 1))
        h = step(g, rows_l, cols_l, counts, inv)
        g = h
    return h[:n]


# speedup vs baseline: 1.0117x; 1.0117x over previous
import functools

import jax
import jax.numpy as jnp
from jax import lax
from jax.experimental import pallas as pl
from jax.experimental.pallas import tpu as pltpu
from jax.experimental.pallas import tpu_sc as plsc

N_NODES = 10000
D_FEAT = 128
K_STEPS = 2
NC = 2
NS = 16
NW = NC * NS
N_PAD = 10240
HALF = N_PAD // NC
PAD_ROWS = N_PAD - N_NODES
ROWS_PER_TILE = N_PAD // NW
ROWS_PER_SUB = N_PAD // NS
SUB_ROWS = HALF // NS
CHUNK = 128
LANES = 16
FILL = 384

_MESH = plsc.VectorSubcoreMesh(core_axis_name="c", subcore_axis_name="s")
_f32 = jnp.float32
_i32 = jnp.int32
_PARAMS = pltpu.CompilerParams(use_tc_tiling_on_sc=False)
_PARAMS_NOLAYOUT = pltpu.CompilerParams(
    use_tc_tiling_on_sc=False, needs_layout_passes=False)


def _part_body(ept, ei_h, ones_h, zeros16_h,
               rows_l, cols_l, counts, degp,
               row1d, col1d, ra, ca, rb, cb, ones_v, cnt_v, deg_acc, sem_d):
    cid = lax.axis_index("c")
    sid = lax.axis_index("s")
    wid = cid * NS + sid
    esl = pl.ds(wid * ept, ept)
    pltpu.sync_copy(ei_h.at[0, esl], row1d)
    pltpu.sync_copy(ei_h.at[1, esl], col1d)
    pltpu.sync_copy(ones_h, ones_v)
    sl = pl.ds(sid * ROWS_PER_SUB, ROWS_PER_SUB)
    pltpu.sync_copy(zeros16_h, deg_acc.at[sl])
    plsc.subcore_barrier()

    nfull = ept // CHUNK
    ntail = (ept % CHUNK) // LANES

    def _fire_d(j):
        pltpu.async_copy(ones_v, deg_acc.at[col1d.at[pl.ds(j * CHUNK, CHUNK)]],
                         sem_d, add=True)

    def _wait_d():
        pltpu.make_async_copy(
            ones_v, deg_acc.at[col1d.at[pl.ds(0, CHUNK)]], sem_d).wait()

    _fire_d(0)
    _fire_d(1)

    def dstep(j, carry):
        _fire_d(j + 2)
        _wait_d()
        return carry

    lax.fori_loop(0, nfull - 2, dstep, 0)
    for t in range(ntail):
        pltpu.async_copy(
            ones_v.at[pl.ds(0, LANES)],
            deg_acc.at[col1d.at[pl.ds(nfull * CHUNK + t * LANES, LANES)]],
            sem_d, add=True)

    def cstep(k, carry):
        offa, offb = carry
        r16 = row1d[pl.ds(k * LANES, LANES)]
        c16 = col1d[pl.ds(k * LANES, LANES)]
        ma = r16 < HALF
        mb = jnp.logical_not(ma)
        pca = plsc.cumsum(ma.astype(_i32))
        posa = jnp.full((LANES,), offa - 1, _i32) + pca
        posb = jnp.full((LANES,), offb, _i32) + (lax.iota(_i32, LANES) - pca)
        plsc.store_scatter(ra, [posa], r16, mask=ma)
        plsc.store_scatter(ca, [posa], c16, mask=ma)
        plsc.store_scatter(rb, [posb], r16 - HALF, mask=mb)
        plsc.store_scatter(cb, [posb], c16, mask=mb)
        na = pca[LANES - 1]
        return offa + na, offb + (LANES - na)

    offa, offb = lax.fori_loop(0, ept // LANES, cstep, (0, 0))
    _wait_d()
    _wait_d()
    for t in range(ntail):
        pltpu.make_async_copy(
            ones_v.at[pl.ds(0, LANES)],
            deg_acc.at[col1d.at[pl.ds(0, LANES)]], sem_d).wait()

    iota = lax.iota(_i32, LANES)
    for k in range(FILL // LANES):
        rd = (iota * 331 + k * LANES) % HALF
        cd = N_NODES + ((iota + k) * 7) % PAD_ROWS
        ra[pl.ds(offa + k * LANES, LANES)] = rd
        ca[pl.ds(offa + k * LANES, LANES)] = cd
        rb[pl.ds(offb + k * LANES, LANES)] = rd
        cb[pl.ds(offb + k * LANES, LANES)] = cd

    def _chunks(off):
        c = (off + CHUNK - 1) // CHUNK
        c = c + (c % 2)
        return jnp.maximum(c, 2)

    cnt_v[...] = jnp.full((LANES,), _chunks(offa), _i32)
    pltpu.sync_copy(cnt_v, counts.at[0, wid])
    cnt_v[...] = jnp.full((LANES,), _chunks(offb), _i32)
    pltpu.sync_copy(cnt_v, counts.at[1, wid])
    pltpu.sync_copy(ra, rows_l.at[0, wid])
    pltpu.sync_copy(ca, cols_l.at[0, wid])
    pltpu.sync_copy(rb, rows_l.at[1, wid])
    pltpu.sync_copy(cb, cols_l.at[1, wid])

    plsc.subcore_barrier()
    pltpu.sync_copy(deg_acc.at[sl], degp.at[cid, sl])


def _make_part(ept):
    cap = ept + FILL
    return pl.kernel(
        functools.partial(_part_body, ept),
        out_type=(
            jax.ShapeDtypeStruct((NC, NW, cap), _i32),
            jax.ShapeDtypeStruct((NC, NW, cap), _i32),
            jax.ShapeDtypeStruct((NC, NW, LANES), _i32),
            jax.ShapeDtypeStruct((NC, N_PAD, LANES), _f32),
        ),
        mesh=_MESH,
        compiler_params=_PARAMS_NOLAYOUT,
        scratch_types=[
            pltpu.VMEM((ept,), _i32),
            pltpu.VMEM((ept,), _i32),
            pltpu.VMEM((cap,), _i32),
            pltpu.VMEM((cap,), _i32),
            pltpu.VMEM((cap,), _i32),
            pltpu.VMEM((cap,), _i32),
            pltpu.VMEM((CHUNK, LANES), _f32),
            pltpu.VMEM((LANES,), _i32),
            pltpu.VMEM_SHARED((N_PAD, LANES), _f32),
            pltpu.SemaphoreType.DMA,
        ],
    )


def _scale_body(x_h, degp_h, g_h, inv_h, x_v, da_v, db_v, g_v, inv_v):
    wid = lax.axis_index("c") * NS + lax.axis_index("s")
    sl = pl.ds(wid * ROWS_PER_TILE, ROWS_PER_TILE)
    pltpu.sync_copy(x_h.at[sl], x_v)
    pltpu.sync_copy(degp_h.at[0, sl], da_v)
    pltpu.sync_copy(degp_h.at[1, sl], db_v)

    def row(r, carry):
        d16 = da_v[r, :] + db_v[r, :] + 1.0
        inv16 = 1.0 / d16
        inv_v[r, :] = inv16
        for f in range(D_FEAT // LANES):
            s = pl.ds(f * LANES, LANES)
            g_v[r, s] = x_v[r, s] * inv16
        return carry

    lax.fori_loop(0, ROWS_PER_TILE, row, 0)
    pltpu.sync_copy(g_v, g_h.at[sl])
    pltpu.sync_copy(inv_v, inv_h.at[sl])


_scale_kernel = pl.kernel(
    _scale_body,
    out_type=(
        jax.ShapeDtypeStruct((N_PAD, D_FEAT), _f32),
        jax.ShapeDtypeStruct((N_PAD, LANES), _f32),
    ),
    mesh=_MESH,
    compiler_params=_PARAMS,
    scratch_types=[
        pltpu.VMEM((ROWS_PER_TILE, D_FEAT), _f32),
        pltpu.VMEM((ROWS_PER_TILE, LANES), _f32),
        pltpu.VMEM((ROWS_PER_TILE, LANES), _f32),
        pltpu.VMEM((ROWS_PER_TILE, D_FEAT), _f32),
        pltpu.VMEM((ROWS_PER_TILE, LANES), _f32),
    ],
)


def _step_body(scale_out, g_h, rows_l, cols_l, counts_h, inv_h, out_h,
               ra, ca, rb, cb, cnt0_v, cnt1_v, inv_v, buf0, buf1, acc,
               s0, s1):
    cid = lax.axis_index("c")
    sid = lax.axis_index("s")
    half0 = cid * HALF
    lsl = pl.ds(sid * SUB_ROWS, SUB_ROWS)
    gsl = pl.ds(half0 + sid * SUB_ROWS, SUB_ROWS)
    pltpu.sync_copy(g_h.at[gsl], acc.at[lsl])
    src0 = sid
    src1 = NS + sid
    pltpu.sync_copy(rows_l.at[cid, src0], ra)
    pltpu.sync_copy(cols_l.at[cid, src0], ca)
    pltpu.sync_copy(rows_l.at[cid, src1], rb)
    pltpu.sync_copy(cols_l.at[cid, src1], cb)
    pltpu.sync_copy(counts_h.at[cid, src0], cnt0_v)
    pltpu.sync_copy(counts_h.at[cid, src1], cnt1_v)
    plsc.subcore_barrier()

    for r_l, c_l, cnt_ref in ((ra, ca, cnt0_v), (rb, cb, cnt1_v)):
        cnt = cnt_ref[...][0]

        def _idx(j):
            return c_l.at[pl.ds(j * CHUNK, CHUNK)]

        def _scatter(j, buf):
            pltpu.sync_copy(buf, acc.at[r_l.at[pl.ds(j * CHUNK, CHUNK)]],
                            add=True)

        pltpu.async_copy(g_h.at[_idx(0)], buf0, s0)
        pltpu.async_copy(g_h.at[_idx(1)], buf1, s1)

        def pair(i, carry):
            j0 = i * 2
            pltpu.make_async_copy(g_h.at[_idx(j0)], buf0, s0).wait()
            _scatter(j0, buf0)
            pltpu.async_copy(g_h.at[_idx(j0 + 2)], buf0, s0)
            pltpu.make_async_copy(g_h.at[_idx(j0 + 1)], buf1, s1).wait()
            _scatter(j0 + 1, buf1)
            pltpu.async_copy(g_h.at[_idx(j0 + 3)], buf1, s1)
            return carry

        lax.fori_loop(0, cnt // 2 - 1, pair, 0)
        pltpu.make_async_copy(g_h.at[_idx(cnt - 2)], buf0, s0).wait()
        _scatter(cnt - 2, buf0)
        pltpu.make_async_copy(g_h.at[_idx(cnt - 1)], buf1, s1).wait()
        _scatter(cnt - 1, buf1)

    plsc.subcore_barrier()
    if not scale_out:
        pltpu.sync_copy(acc.at[lsl], out_h.at[gsl])
    else:
        pltpu.sync_copy(inv_h.at[gsl], inv_v)
        done = 0
        while done < SUB_ROWS:
            cn = min(CHUNK, SUB_ROWS - done)
            pltpu.sync_copy(acc.at[pl.ds(sid * SUB_ROWS + done, cn)],
                            buf0.at[pl.ds(0, cn)])

            def rower(r, carry):
                inv16 = inv_v[done + r, :]
                for f in range(D_FEAT // LANES):
                    s = pl.ds(f * LANES, LANES)
                    buf0[r, s] = buf0[r, s] * inv16
                return carry

            lax.fori_loop(0, cn, rower, 0)
            pltpu.sync_copy(
                buf0.at[pl.ds(0, cn)],
                out_h.at[pl.ds(half0 + sid * SUB_ROWS + done, cn)])
            done += cn


def _make_step(ept, scale_out):
    cap = ept + FILL
    return pl.kernel(
        functools.partial(_step_body, scale_out),
        out_type=jax.ShapeDtypeStruct((N_PAD, D_FEAT), _f32),
        mesh=_MESH,
        compiler_params=_PARAMS,
        scratch_types=[
            pltpu.VMEM((cap,), _i32),
            pltpu.VMEM((cap,), _i32),
            pltpu.VMEM((cap,), _i32),
            pltpu.VMEM((cap,), _i32),
            pltpu.VMEM((LANES,), _i32),
            pltpu.VMEM((LANES,), _i32),
            pltpu.VMEM((SUB_ROWS, LANES), _f32),
            pltpu.VMEM((CHUNK, D_FEAT), _f32),
            pltpu.VMEM((CHUNK, D_FEAT), _f32),
            pltpu.VMEM_SHARED((HALF, D_FEAT), _f32),
            pltpu.SemaphoreType.DMA,
            pltpu.SemaphoreType.DMA,
        ],
    )


def kernel(x, edge_index):
    n, d = x.shape
    e = edge_index.shape[1]
    assert n == N_NODES and d == D_FEAT

    ei = edge_index.astype(_i32)

    align = NW * LANES
    if e % align:
        flat_pad = align - e % align
        dummy = N_NODES + (jnp.arange(flat_pad, dtype=_i32) % PAD_ROWS)
        ei = jnp.concatenate([ei, jnp.stack([dummy, dummy])], axis=1)
        e += flat_pad
    ept = e // NW

    xpad = jnp.zeros((N_PAD, D_FEAT), _f32).at[:n].set(x.astype(_f32))
    zeros16 = jnp.zeros((ROWS_PER_SUB, LANES), _f32)
    ones = jnp.ones((CHUNK, LANES), _f32)

    rows_l, cols_l, counts, degp = _make_part(ept)(ei, ones, zeros16)
    g, inv = _scale_kernel(xpad, degp)
    h = None
    for k in range(K_STEPS):
        step = _make_step(ept, scale_out=(k < K_STEPS - 1))
        h = step(g, rows_l, cols_l, counts, inv)
        g = h
    return h[:n]
